# baseline (speedup 1.0000x reference)
import jax
import jax.numpy as jnp
from jax.experimental import pallas as pl
from jax import lax

D_MODEL = 768
NUM_EXPERTS = 8
TOP_K = 2

_ROWS_PER_BLOCK = 2048


def _router_block(x_ref, wt_ref, idx_ref, w_ref, logits_ref):
    x = x_ref[...]
    wt = wt_ref[...]
    logits = jnp.dot(x, wt, preferred_element_type=jnp.float32)

    m = jnp.max(logits, axis=-1, keepdims=True)
    e = jnp.exp(logits - m)
    z = jnp.sum(e, axis=-1, keepdims=True)
    probs = e / z

    iota = lax.broadcasted_iota(jnp.int32, probs.shape, 1)
    big = jnp.int32(NUM_EXPERTS)

    v1 = jnp.max(probs, axis=-1, keepdims=True)
    i1 = jnp.min(jnp.where(probs == v1, iota, big), axis=-1, keepdims=True)
    masked = jnp.where(iota == i1, -jnp.inf, probs)
    v2 = jnp.max(masked, axis=-1, keepdims=True)
    i2 = jnp.min(jnp.where(masked == v2, iota, big), axis=-1, keepdims=True)

    denom = v1 + v2 + 1e-09
    w_ref[...] = jnp.concatenate([v1 / denom, v2 / denom], axis=-1)
    idx_ref[...] = jnp.concatenate([i1, i2], axis=-1)
    logits_ref[...] = logits


def kernel(x, W):
    b, s, d = x.shape
    n_rows = b * s
    xf = x.reshape(n_rows, d)
    wt = W.T

    grid = (n_rows // _ROWS_PER_BLOCK,)
    r = _ROWS_PER_BLOCK

    idx, w, logits = pl.pallas_call(
        _router_block,
        grid=grid,
        in_specs=[
            pl.BlockSpec((r, d), lambda i: (i, 0)),
            pl.BlockSpec((d, NUM_EXPERTS), lambda i: (0, 0)),
        ],
        out_specs=[
            pl.BlockSpec((r, TOP_K), lambda i: (i, 0)),
            pl.BlockSpec((r, TOP_K), lambda i: (i, 0)),
            pl.BlockSpec((r, NUM_EXPERTS), lambda i: (i, 0)),
        ],
        out_shape=[
            jax.ShapeDtypeStruct((n_rows, TOP_K), jnp.int32),
            jax.ShapeDtypeStruct((n_rows, TOP_K), jnp.float32),
            jax.ShapeDtypeStruct((n_rows, NUM_EXPERTS), jnp.float32),
        ],
    )(xf, wt)

    return (
        idx.reshape(b, s, TOP_K),
        w.reshape(b, s, TOP_K),
        logits.reshape(b, s, NUM_EXPERTS),
    )
